# Initial kernel scaffold; baseline (speedup 1.0000x reference)
import jax
import jax.numpy as jnp
from jax.experimental import pallas as pl


def kernel(x_nchw, w_oihw, bias, gamma, beta):
    raise NotImplementedError("write your pallas kernel here")


# speedup vs baseline: 1.5294x; 1.5294x over previous
import functools

import jax
import jax.numpy as jnp
from jax.experimental import pallas as pl
from jax.experimental.pallas import tpu as pltpu

EPS = 1e-5
NEG_SLOPE = 0.2


def _conv_stats_kernel(x_ref, w_ref, b_ref, y_ref, s_ref, patch_ref, *,
                       cin, cout, h, w):
    hw = h * w
    x = x_ref[0].astype(jnp.bfloat16)

    col = jax.lax.broadcasted_iota(jnp.int32, (1, hw), 1) % w

    tap = 0
    for oy in (-1, 0, 1):
        for ox in (-1, 0, 1):
            s = oy * w + ox
            if s > 0:
                sh = jnp.concatenate(
                    [x[:, s:], jnp.zeros((cin, s), jnp.bfloat16)], axis=1)
            elif s < 0:
                sh = jnp.concatenate(
                    [jnp.zeros((cin, -s), jnp.bfloat16), x[:, :s]], axis=1)
            else:
                sh = x
            if ox == -1:
                sh = jnp.where(col != 0, sh, jnp.bfloat16(0))
            elif ox == 1:
                sh = jnp.where(col != w - 1, sh, jnp.bfloat16(0))
            patch_ref[pl.ds(tap * cin, cin), :] = sh
            tap += 1

    acc = jnp.dot(w_ref[...], patch_ref[...],
                  preferred_element_type=jnp.float32)
    y = acc + b_ref[...]
    y_ref[0] = y.astype(jnp.bfloat16)

    s_ref[0] = jnp.concatenate(
        [jnp.sum(y, axis=1, keepdims=True),
         jnp.sum(y * y, axis=1, keepdims=True),
         jnp.zeros((cout, 126), jnp.float32)], axis=1)


def _bn_lrelu_kernel(y_ref, sc_ref, sh_ref, o_ref):
    y = y_ref[0].astype(jnp.float32)
    out = y * sc_ref[...] + sh_ref[...]
    o_ref[0] = jnp.where(out >= 0, out, NEG_SLOPE * out)


@jax.jit
def _forward(x_nchw, w_oihw, bias, gamma, beta):
    N, Cin, H, W = x_nchw.shape
    Cout = w_oihw.shape[0]
    HW = H * W

    x3 = x_nchw.reshape(N, Cin, HW)
    w_slab = jnp.transpose(w_oihw, (0, 2, 3, 1)).reshape(Cout, 9 * Cin)
    w_slab = w_slab.astype(jnp.bfloat16)
    b_col = bias.astype(jnp.float32).reshape(Cout, 1)

    conv_kernel = functools.partial(
        _conv_stats_kernel, cin=Cin, cout=Cout, h=H, w=W)

    y3, stats = pl.pallas_call(
        conv_kernel,
        out_shape=(
            jax.ShapeDtypeStruct((N, Cout, HW), jnp.bfloat16),
            jax.ShapeDtypeStruct((N, Cout, 128), jnp.float32),
        ),
        grid=(N,),
        in_specs=[
            pl.BlockSpec((1, Cin, HW), lambda n: (n, 0, 0)),
            pl.BlockSpec((Cout, 9 * Cin), lambda n: (0, 0)),
            pl.BlockSpec((Cout, 1), lambda n: (0, 0)),
        ],
        out_specs=(
            pl.BlockSpec((1, Cout, HW), lambda n: (n, 0, 0)),
            pl.BlockSpec((1, Cout, 128), lambda n: (n, 0, 0)),
        ),
        scratch_shapes=[
            pltpu.VMEM((9 * Cin, HW), jnp.bfloat16),
        ],
        compiler_params=pltpu.CompilerParams(
            dimension_semantics=("parallel",)),
    )(x3, w_slab, b_col)

    totals = jnp.sum(stats, axis=0)
    count = jnp.float32(N * HW)
    mean = totals[:, 0] / count
    var = jnp.maximum(totals[:, 1] / count - mean * mean, 0.0)
    inv_std = jax.lax.rsqrt(var + EPS)
    g = gamma.astype(jnp.float32)
    scale = (g * inv_std).reshape(Cout, 1)
    shift = (beta.astype(jnp.float32) - mean * g * inv_std).reshape(Cout, 1)

    out3 = pl.pallas_call(
        _bn_lrelu_kernel,
        out_shape=jax.ShapeDtypeStruct((N, Cout, HW), jnp.float32),
        grid=(N,),
        in_specs=[
            pl.BlockSpec((1, Cout, HW), lambda n: (n, 0, 0)),
            pl.BlockSpec((Cout, 1), lambda n: (0, 0)),
            pl.BlockSpec((Cout, 1), lambda n: (0, 0)),
        ],
        out_specs=pl.BlockSpec((1, Cout, HW), lambda n: (n, 0, 0)),
        compiler_params=pltpu.CompilerParams(
            dimension_semantics=("parallel",)),
    )(y3, scale, shift)

    return out3.reshape(N, Cout, H, W).astype(x_nchw.dtype)


def kernel(x_nchw, w_oihw, bias, gamma, beta):
    return _forward(x_nchw, w_oihw, bias, gamma, beta)
